# baseline (speedup 1.0000x reference)
import functools

import jax
import jax.numpy as jnp
from jax import lax
from jax.experimental import pallas as pl
from jax.experimental.pallas import tpu as pltpu
from jax.experimental.pallas import tpu_sc as plsc

_NUM_CORES = 2
_NUM_SUBCORES = 16
_NUM_WORKERS = _NUM_CORES * _NUM_SUBCORES

_CHUNK = 512
_NBUF = 2


def _make_lookup(num_rows: int, dim: int, batch: int):
  assert batch % _NUM_WORKERS == 0
  b_per_w = batch // _NUM_WORKERS
  assert b_per_w % _CHUNK == 0
  n_chunks = b_per_w // _CHUNK
  assert n_chunks % _NBUF == 0
  n_groups = n_chunks // _NBUF

  mesh = plsc.VectorSubcoreMesh(
      core_axis_name="c", subcore_axis_name="s", num_cores=_NUM_CORES)

  @functools.partial(
      pl.kernel,
      mesh=mesh,
      compiler_params=pltpu.CompilerParams(use_tc_tiling_on_sc=False),
      out_type=jax.ShapeDtypeStruct((batch, dim), jnp.float32),
      scratch_types=[
          pltpu.VMEM((b_per_w,), jnp.int32),
          pltpu.VMEM((_NBUF, _CHUNK, dim), jnp.float32),
          pltpu.SemaphoreType.DMA,
          pltpu.SemaphoreType.DMA,
      ],
  )
  def lookup(table_hbm, idx_hbm, out_hbm, idx_v, rows_v, gsem0, gsem1):
    gsems = (gsem0, gsem1)
    wid = lax.axis_index("s") * _NUM_CORES + lax.axis_index("c")
    base = wid * b_per_w
    pltpu.sync_copy(idx_hbm.at[pl.ds(base, b_per_w)], idx_v)

    @pl.loop(0, n_groups)
    def _group(g):
      descs = []
      for b in range(_NBUF):
        off = pl.multiple_of((g * _NBUF + b) * _CHUNK, _CHUNK)
        descs.append(
            pltpu.async_copy(
                table_hbm.at[idx_v.at[pl.ds(off, _CHUNK)]],
                rows_v.at[b], gsems[b]))
      for b in range(_NBUF):
        off = pl.multiple_of((g * _NBUF + b) * _CHUNK, _CHUNK)
        descs[b].wait()
        pltpu.sync_copy(rows_v.at[b], out_hbm.at[pl.ds(base + off, _CHUNK)])

  return lookup


def kernel(token_ids, embedding_matrix):
  s0, s1 = token_ids.shape
  num_rows, dim = embedding_matrix.shape
  batch = s0 * s1
  idx = token_ids.reshape(batch).astype(jnp.int32)
  lookup = _make_lookup(num_rows, dim, batch)
  out = lookup(embedding_matrix, idx)
  return out.reshape(s0, s1, dim)


# speedup vs baseline: 1.0031x; 1.0031x over previous
import functools

import jax
import jax.numpy as jnp
from jax import lax
from jax.experimental import pallas as pl
from jax.experimental.pallas import tpu as pltpu
from jax.experimental.pallas import tpu_sc as plsc

_NUM_CORES = 2
_NUM_SUBCORES = 16
_NUM_WORKERS = _NUM_CORES * _NUM_SUBCORES

_GRP = 4
_NBUF = 2


def _make_lookup(num_rows: int, dim: int, s0: int, s1: int):
  assert s0 % _NUM_WORKERS == 0
  rows_per_w = s0 // _NUM_WORKERS
  assert rows_per_w % (_GRP * _NBUF) == 0
  n_groups = rows_per_w // (_GRP * _NBUF)
  assert s1 % 8 == 0

  mesh = plsc.VectorSubcoreMesh(
      core_axis_name="c", subcore_axis_name="s", num_cores=_NUM_CORES)

  @functools.partial(
      pl.kernel,
      mesh=mesh,
      compiler_params=pltpu.CompilerParams(use_tc_tiling_on_sc=False),
      out_type=jax.ShapeDtypeStruct((s0, s1, dim), jnp.float32),
      scratch_types=[
          pltpu.VMEM((rows_per_w, s1), jnp.int32),
          pltpu.VMEM((_NBUF, _GRP, s1, dim), jnp.float32),
          pltpu.SemaphoreType.DMA,
          pltpu.SemaphoreType.DMA,
      ],
  )
  def lookup(table_hbm, idx_hbm, out_hbm, idx_v, rows_v, gsem0, gsem1):
    gsems = (gsem0, gsem1)
    wid = lax.axis_index("s") * _NUM_CORES + lax.axis_index("c")
    wbase = wid * rows_per_w
    pltpu.sync_copy(idx_hbm.at[pl.ds(wbase, rows_per_w)], idx_v)

    @pl.loop(0, n_groups)
    def _group(g):
      descs = [[] for _ in range(_NBUF)]
      for b in range(_NBUF):
        for j in range(_GRP):
          row = (g * _NBUF + b) * _GRP + j
          descs[b].append(
              pltpu.async_copy(
                  table_hbm.at[idx_v.at[row]],
                  rows_v.at[b].at[j], gsems[b]))
      for b in range(_NBUF):
        for d in descs[b]:
          d.wait()
        row0 = (g * _NBUF + b) * _GRP
        pltpu.sync_copy(rows_v.at[b], out_hbm.at[pl.ds(wbase + row0, _GRP)])

  return lookup


def kernel(token_ids, embedding_matrix):
  s0, s1 = token_ids.shape
  num_rows, dim = embedding_matrix.shape
  idx = token_ids.astype(jnp.int32)
  lookup = _make_lookup(num_rows, dim, s0, s1)
  return lookup(embedding_matrix, idx)


# speedup vs baseline: 1.2191x; 1.2154x over previous
import functools

import jax
import jax.numpy as jnp
from jax import lax
from jax.experimental import pallas as pl
from jax.experimental.pallas import tpu as pltpu
from jax.experimental.pallas import tpu_sc as plsc

_NUM_CORES = 2
_NUM_SUBCORES = 16
_NUM_WORKERS = _NUM_CORES * _NUM_SUBCORES
_PAD = 128

_GRP = 2
_NBUF = 2


def _make_lookup(num_rows: int, dim: int, s0: int, s1: int):
  assert s0 % _NUM_WORKERS == 0
  rows_per_w = s0 // _NUM_WORKERS
  assert rows_per_w % (_GRP * _NBUF) == 0
  n_groups = rows_per_w // (_GRP * _NBUF)
  assert s1 % 8 == 0

  mesh = plsc.VectorSubcoreMesh(
      core_axis_name="c", subcore_axis_name="s", num_cores=_NUM_CORES)

  @functools.partial(
      pl.kernel,
      mesh=mesh,
      compiler_params=pltpu.CompilerParams(use_tc_tiling_on_sc=False),
      out_type=jax.ShapeDtypeStruct((s0, s1, _PAD), jnp.float32),
      scratch_types=[
          pltpu.VMEM((rows_per_w, s1), jnp.int32),
          pltpu.VMEM((_NBUF, _GRP, s1, _PAD), jnp.float32),
          pltpu.SemaphoreType.DMA,
          pltpu.SemaphoreType.DMA,
      ],
  )
  def lookup(table_hbm, idx_hbm, out_hbm, idx_v, rows_v, gsem0, gsem1):
    gsems = (gsem0, gsem1)
    wid = lax.axis_index("s") * _NUM_CORES + lax.axis_index("c")
    wbase = wid * rows_per_w
    pltpu.sync_copy(idx_hbm.at[pl.ds(wbase, rows_per_w)], idx_v)

    @pl.loop(0, n_groups)
    def _group(g):
      descs = [[] for _ in range(_NBUF)]
      for b in range(_NBUF):
        for j in range(_GRP):
          row = (g * _NBUF + b) * _GRP + j
          descs[b].append(
              pltpu.async_copy(
                  table_hbm.at[idx_v.at[row]],
                  rows_v.at[b].at[j], gsems[b]))
      for b in range(_NBUF):
        for d in descs[b]:
          d.wait()
        row0 = (g * _NBUF + b) * _GRP
        pltpu.sync_copy(
            rows_v.at[b].at[:, :, pl.ds(0, dim)],
            out_hbm.at[pl.ds(wbase + row0, _GRP), :, pl.ds(0, dim)])

  return lookup


def kernel(token_ids, embedding_matrix):
  s0, s1 = token_ids.shape
  num_rows, dim = embedding_matrix.shape
  idx = token_ids.astype(jnp.int32)
  tbl = jnp.pad(embedding_matrix, ((0, 0), (0, _PAD - dim)))
  lookup = _make_lookup(num_rows, dim, s0, s1)
  padded = lookup(tbl, idx)
  return padded[:, :, :dim]


# speedup vs baseline: 1.4314x; 1.1742x over previous
import functools

import jax
import jax.numpy as jnp
from jax import lax
from jax.experimental import pallas as pl
from jax.experimental.pallas import tpu as pltpu
from jax.experimental.pallas import tpu_sc as plsc

_NUM_CORES = 2
_NUM_SUBCORES = 16
_NUM_WORKERS = _NUM_CORES * _NUM_SUBCORES
_PAD = 128

_GRP = 4
_NBUF = 2


def _make_lookup(num_rows: int, dim: int, s0: int, s1: int):
  assert s0 % _NUM_WORKERS == 0
  rows_per_w = s0 // _NUM_WORKERS
  assert rows_per_w % (_GRP * _NBUF) == 0
  n_groups = rows_per_w // (_GRP * _NBUF)
  assert s1 % 8 == 0

  mesh = plsc.VectorSubcoreMesh(
      core_axis_name="c", subcore_axis_name="s", num_cores=_NUM_CORES)

  @functools.partial(
      pl.kernel,
      mesh=mesh,
      compiler_params=pltpu.CompilerParams(use_tc_tiling_on_sc=False),
      out_type=jax.ShapeDtypeStruct((s0, s1, _PAD), jnp.float32),
      scratch_types=[
          pltpu.VMEM((rows_per_w, s1), jnp.int32),
          pltpu.VMEM((_NBUF, _GRP, s1, dim), jnp.float32),
          pltpu.SemaphoreType.DMA,
          pltpu.SemaphoreType.DMA,
      ],
  )
  def lookup(table_hbm, idx_hbm, out_hbm, idx_v, rows_v, gsem0, gsem1):
    gsems = (gsem0, gsem1)
    wid = lax.axis_index("s") * _NUM_CORES + lax.axis_index("c")
    wbase = wid * rows_per_w
    pltpu.sync_copy(idx_hbm.at[pl.ds(wbase, rows_per_w)], idx_v)

    @pl.loop(0, n_groups)
    def _group(g):
      descs = [[] for _ in range(_NBUF)]
      for b in range(_NBUF):
        for j in range(_GRP):
          row = (g * _NBUF + b) * _GRP + j
          descs[b].append(
              pltpu.async_copy(
                  table_hbm.at[idx_v.at[row]],
                  rows_v.at[b].at[j], gsems[b]))
      for b in range(_NBUF):
        for d in descs[b]:
          d.wait()
        row0 = (g * _NBUF + b) * _GRP
        pltpu.sync_copy(
            rows_v.at[b],
            out_hbm.at[pl.ds(wbase + row0, _GRP), :, pl.ds(0, dim)])

  return lookup


def kernel(token_ids, embedding_matrix):
  s0, s1 = token_ids.shape
  num_rows, dim = embedding_matrix.shape
  idx2 = token_ids.astype(jnp.int32) * 2
  fill = jnp.zeros((num_rows, _PAD - dim), jnp.float32)
  tbl = jnp.concatenate([embedding_matrix, fill], axis=1)
  tbl2 = tbl.reshape(num_rows * 2, dim)
  lookup = _make_lookup(num_rows, dim, s0, s1)
  padded = lookup(tbl2, idx2)
  return padded[:, :, :dim]
